# baseline (speedup 1.0000x reference)
import dataclasses
import functools

import jax
import jax.numpy as jnp
from jax import lax
from jax.experimental import pallas as pl
from jax.experimental.pallas import tpu as pltpu
from jax.experimental.pallas import tpu_sc as plsc

H, W = 480, 640
HW = H * W
LANES = 16


def _build_gather(B, N, num_workers, chunk):
    total = B * N
    per_worker = total // num_workers
    nchunks = per_worker // chunk
    mesh = plsc.VectorSubcoreMesh(core_axis_name="c", subcore_axis_name="s")
    num_cores = 2
    cp = pltpu.CompilerParams()
    if "needs_layout_passes" in pltpu.CompilerParams.__dataclass_fields__:
        cp = dataclasses.replace(cp, needs_layout_passes=False)

    @functools.partial(
        pl.kernel,
        out_type=jax.ShapeDtypeStruct((B * 2 * N,), jnp.float32),
        mesh=mesh,
        compiler_params=cp,
        scratch_types=[
            pltpu.VMEM((chunk * 4,), jnp.int32),
            pltpu.VMEM((2 * chunk,), jnp.int32),
            pltpu.VMEM((2 * chunk,), jnp.float32),
            pltpu.SemaphoreType.DMA,
        ],
    )
    def gather_kernel(flow_hbm, ev_hbm, out_hbm, evbuf, gidx, obuf, sem):
        cid = lax.axis_index("c")
        sid = lax.axis_index("s")
        wid = sid * num_cores + cid
        base = wid * per_worker
        lanes = lax.iota(jnp.int32, LANES)

        @pl.loop(0, nchunks)
        def _chunk(c):
            g = base + c * chunk
            pltpu.sync_copy(ev_hbm.at[pl.ds(g * 4, chunk * 4)], evbuf)
            bofs = (g // N) * (2 * HW)

            @pl.loop(0, chunk, step=LANES)
            def _vec(t):
                pos = lanes + t
                mask = pos < chunk
                y = plsc.load_gather(evbuf, [pos * 4 + 1], mask=mask)
                x = plsc.load_gather(evbuf, [pos * 4 + 2], mask=mask)
                fidx = y * W + x + bofs
                plsc.store_scatter(gidx, [pos * 2], fidx + HW, mask=mask)
                plsc.store_scatter(gidx, [pos * 2 + 1], fidx, mask=mask)

            pltpu.async_copy(flow_hbm.at[gidx], obuf, sem).wait()
            pltpu.sync_copy(obuf, out_hbm.at[pl.ds(g * 2, chunk * 2)])

    return gather_kernel


def kernel(flow, event_list, pol_mask, event_mask):
    B, _, h, w = flow.shape
    N = event_list.shape[1]
    flow_flat = flow.reshape(B * 2 * HW)
    ev_flat = event_list.reshape(B * N * 4)
    gk = _build_gather(B, N, num_workers=32, chunk=2500)
    out_flat = gk(flow_flat, ev_flat)
    return out_flat.reshape(B, N, 2)


# speedup vs baseline: 13.5774x; 13.5774x over previous
import dataclasses
import functools

import jax
import jax.numpy as jnp
from jax import lax
from jax.experimental import pallas as pl
from jax.experimental.pallas import tpu as pltpu
from jax.experimental.pallas import tpu_sc as plsc

H, W = 480, 640
HW = H * W
LANES = 16
LANE_TILE = 128
PLANE = HW
WTILES = W // LANE_TILE
NUM_WORKERS = 32
NUM_CORES = 2


def _build_gather(B, N):
    ntiles_b = (N + LANE_TILE - 1) // LANE_TILE
    et_total = B * ntiles_b
    n_pad = ntiles_b * LANE_TILE
    slots = et_total * 2 * LANE_TILE

    tq, rem = divmod(et_total, NUM_WORKERS)
    CT = 49
    NCH = -(-(tq + 1) // CT)
    CHUNK_WORDS = CT * 2 * LANE_TILE

    mesh = plsc.VectorSubcoreMesh(core_axis_name="c", subcore_axis_name="s")
    cp = pltpu.CompilerParams()
    if "needs_layout_passes" in pltpu.CompilerParams.__dataclass_fields__:
        cp = dataclasses.replace(cp, needs_layout_passes=False)

    @functools.partial(
        pl.kernel,
        out_type=jax.ShapeDtypeStruct((slots,), jnp.float32),
        mesh=mesh,
        compiler_params=cp,
        scratch_types=[
            pltpu.VMEM((CHUNK_WORDS,), jnp.int32),
            pltpu.VMEM((CHUNK_WORDS,), jnp.float32),
            pltpu.SemaphoreType.DMA,
        ],
    )
    def gather_kernel(flow_hbm, yx_hbm, out_hbm, idxbuf, obuf, sem):
        cid = lax.axis_index("c")
        sid = lax.axis_index("s")
        wid = sid * NUM_CORES + cid
        t0 = wid * tq + jnp.minimum(wid, rem)
        t1 = t0 + tq + jnp.where(wid < rem, 1, 0)

        @pl.loop(0, NCH)
        def _chunk(ci):
            start = jnp.minimum(t0 + ci * CT, t1 - CT)
            base = start * (2 * LANE_TILE)
            pltpu.sync_copy(yx_hbm.at[pl.ds(base, CHUNK_WORDS)], idxbuf)

            @pl.loop(0, CT)
            def _tile(t):
                et = start + t
                b = et // ntiles_b
                plane0 = b * (2 * PLANE)
                toff = t * (2 * LANE_TILE)

                @pl.loop(0, LANE_TILE, step=LANES)
                def _vec(j):
                    yv = idxbuf[pl.ds(toff + j, LANES)]
                    xv = idxbuf[pl.ds(toff + LANE_TILE + j, LANES)]
                    pidx = (
                        ((yv >> 3) * WTILES + (xv >> 7)) * 1024
                        + ((yv & 7) << 7)
                        + (xv & 127)
                    )
                    idxbuf[pl.ds(toff + j, LANES)] = pidx + (plane0 + PLANE)
                    idxbuf[pl.ds(toff + LANE_TILE + j, LANES)] = pidx + plane0

            pltpu.async_copy(flow_hbm.at[idxbuf], obuf, sem).wait()
            pltpu.sync_copy(obuf, out_hbm.at[pl.ds(base, CHUNK_WORDS)])

    return gather_kernel, ntiles_b, n_pad, slots


def kernel(flow, event_list, pol_mask, event_mask):
    B, _, h, w = flow.shape
    N = event_list.shape[1]
    gk, ntiles_b, n_pad, slots = _build_gather(B, N)

    flow_view = (
        flow.reshape(B, 2, H // 8, 8, W // LANE_TILE, LANE_TILE)
        .transpose(0, 1, 2, 4, 3, 5)
        .reshape(-1)
    )

    yc = event_list[:, :, 1]
    xc = event_list[:, :, 2]
    pad = ((0, 0), (0, n_pad - N))
    yp = jnp.pad(yc, pad).reshape(B, ntiles_b, LANE_TILE)
    xp = jnp.pad(xc, pad).reshape(B, ntiles_b, LANE_TILE)
    yx = jnp.stack([yp, xp], axis=2).reshape(-1)

    out1d = gk(flow_view, yx)

    out = (
        out1d.reshape(B, ntiles_b, 2, LANE_TILE)
        .transpose(0, 1, 3, 2)
        .reshape(B, n_pad, 2)[:, :N, :]
    )
    return out
